# Initial kernel scaffold; baseline (speedup 1.0000x reference)
import jax
import jax.numpy as jnp
from jax.experimental import pallas as pl


def kernel(sid_idx, aid_idx, symbol_table, action_table, key_table):
    raise NotImplementedError("write your pallas kernel here")


# speedup vs baseline: 1.0363x; 1.0363x over previous
import functools

import jax
import jax.numpy as jnp
from jax import lax
from jax.experimental import pallas as pl
from jax.experimental.pallas import tpu as pltpu
from jax.experimental.pallas import tpu_sc as plsc

_LANES = 16


def _row_offsets(d):
    offs = list(range(0, d - _LANES + 1, _LANES))
    if offs[-1] + _LANES < d:
        offs.append(d - _LANES)
    return offs


def _grammar_emb_call(N, D, DP, n_workers, n_cores, chunk):
    b_per_w = N // n_workers
    n_chunks = b_per_w // chunk
    DO = 3 * D
    offs = _row_offsets(D)
    mesh = plsc.VectorSubcoreMesh(core_axis_name="c", subcore_axis_name="s")

    @functools.partial(
        pl.kernel,
        mesh=mesh,
        out_type=jax.ShapeDtypeStruct((N * DO,), jnp.float32),
        scratch_types=[
            pltpu.VMEM((n_chunks, 2 * chunk), jnp.int32),
            pltpu.VMEM((2 * chunk, DP), jnp.float32),
            pltpu.VMEM((chunk * DO,), jnp.float32),
            pltpu.VMEM((DP,), jnp.float32),
            pltpu.SemaphoreType.DMA,
        ],
        compiler_params=pltpu.CompilerParams(use_tc_tiling_on_sc=False),
    )
    def grammar_emb(cidx_hbm, table_hbm, key_hbm, out_hbm,
                    cidx_v, gbuf, pbuf, kbuf, sem):
        wid = lax.axis_index("s") * n_cores + lax.axis_index("c")
        base = wid * b_per_w

        pltpu.sync_copy(cidx_hbm.at[wid], cidx_v)

        pltpu.sync_copy(key_hbm, kbuf)
        kvecs = [kbuf[pl.ds(o, _LANES)] for o in offs]

        def fill_key(r, carry):
            dst = r * DO + 2 * D
            for o, kv in zip(offs, kvecs):
                pbuf[pl.ds(dst + o, _LANES)] = kv
            return carry

        lax.fori_loop(0, chunk, fill_key, 0)

        def step(g, carry):
            pltpu.async_copy(table_hbm.at[cidx_v.at[g]], gbuf, sem).wait()

            def repack(r, c):
                dst = r * DO
                for o in offs:
                    pbuf[pl.ds(dst + o, _LANES)] = \
                        gbuf[2 * r, pl.ds(o, _LANES)]
                for o in offs:
                    pbuf[pl.ds(dst + D + o, _LANES)] = \
                        gbuf[2 * r + 1, pl.ds(o, _LANES)]
                return c

            lax.fori_loop(0, chunk, repack, 0)
            pltpu.sync_copy(
                pbuf, out_hbm.at[pl.ds((base + g * chunk) * DO, chunk * DO)])
            return carry

        lax.fori_loop(0, n_chunks, step, 0)

    return grammar_emb


def kernel(sid_idx, aid_idx, symbol_table, action_table, key_table):
    B, S = sid_idx.shape
    D = symbol_table.shape[1]
    V = symbol_table.shape[0]
    N = B * S
    DP = (D + 7) // 8 * 8

    info = plsc.get_sparse_core_info()
    n_workers = info.num_cores * info.num_subcores
    chunk = 64
    b_per_w = N // n_workers
    n_chunks = b_per_w // chunk

    table = jnp.concatenate([symbol_table, action_table], axis=0)
    table_p = jnp.pad(table, ((0, 0), (0, DP - D)))
    key_p = jnp.pad(key_table[0], (0, DP - D))
    cidx = jnp.stack([sid_idx.reshape(N), aid_idx.reshape(N) + V], axis=-1)
    cidx = cidx.reshape(n_workers, n_chunks, 2 * chunk)

    fn = _grammar_emb_call(N, D, DP, n_workers, info.num_cores, chunk)
    out = fn(cidx, table_p, key_p)
    return out.reshape(B, S, 3 * D)


# speedup vs baseline: 1.0905x; 1.0523x over previous
import functools

import jax
import jax.numpy as jnp
from jax import lax
from jax.experimental import pallas as pl
from jax.experimental.pallas import tpu as pltpu
from jax.experimental.pallas import tpu_sc as plsc

_LANES = 16


def _row_offsets(d):
    offs = list(range(0, d - _LANES + 1, _LANES))
    if offs[-1] + _LANES < d:
        offs.append(d - _LANES)
    return offs


def _grammar_emb_call(N, D, DP, n_workers, n_cores, chunk):
    b_per_w = N // n_workers
    n_chunks = b_per_w // chunk
    n_pairs = n_chunks // 2
    DO = 3 * D
    offs = _row_offsets(D)
    mesh = plsc.VectorSubcoreMesh(core_axis_name="c", subcore_axis_name="s")

    @functools.partial(
        pl.kernel,
        mesh=mesh,
        out_type=jax.ShapeDtypeStruct((N * DO,), jnp.float32),
        scratch_types=[
            pltpu.VMEM((b_per_w,), jnp.int32),
            pltpu.VMEM((b_per_w,), jnp.int32),
            pltpu.VMEM((chunk, DP), jnp.float32),
            pltpu.VMEM((chunk, DP), jnp.float32),
            pltpu.VMEM((chunk * DO,), jnp.float32),
            pltpu.VMEM((chunk * DO,), jnp.float32),
            pltpu.VMEM((DP,), jnp.float32),
            pltpu.SemaphoreType.DMA,
            pltpu.SemaphoreType.DMA,
            pltpu.SemaphoreType.DMA,
            pltpu.SemaphoreType.DMA,
        ],
        compiler_params=pltpu.CompilerParams(use_tc_tiling_on_sc=False),
    )
    def grammar_emb(sid_hbm, aid_hbm, sym_hbm, act_hbm, key_hbm, out_hbm,
                    sid_v, aid_v, gs, ga, pbuf0, pbuf1, kbuf,
                    sem_s, sem_a, osem0, osem1):
        wid = lax.axis_index("s") * n_cores + lax.axis_index("c")
        base = wid * b_per_w
        pbufs = (pbuf0, pbuf1)
        osems = (osem0, osem1)

        pltpu.sync_copy(sid_hbm.at[pl.ds(base, b_per_w)], sid_v)
        pltpu.sync_copy(aid_hbm.at[pl.ds(base, b_per_w)], aid_v)

        pltpu.sync_copy(key_hbm, kbuf)
        kvecs = [kbuf[pl.ds(o, _LANES)] for o in offs]

        def fill_key(r, carry):
            dst = r * DO + 2 * D
            for pb in pbufs:
                for o, kv in zip(offs, kvecs):
                    pb[pl.ds(dst + o, _LANES)] = kv
            return carry

        lax.fori_loop(0, chunk, fill_key, 0)

        def half_step(i, p):
            g = 2 * i + p
            r0 = g * chunk
            cp_s = pltpu.async_copy(
                sym_hbm.at[sid_v.at[pl.ds(r0, chunk)]], gs, sem_s)
            cp_a = pltpu.async_copy(
                act_hbm.at[aid_v.at[pl.ds(r0, chunk)]], ga, sem_a)
            cp_s.wait()
            cp_a.wait()

            pb = pbufs[p]

            @pl.when(i >= 1)
            def _():
                pltpu.make_async_copy(
                    pb, out_hbm.at[pl.ds(base * DO, chunk * DO)],
                    osems[p]).wait()

            def repack(r, c):
                dst = r * DO
                for o in offs:
                    pb[pl.ds(dst + o, _LANES)] = gs[r, pl.ds(o, _LANES)]
                for o in offs:
                    pb[pl.ds(dst + D + o, _LANES)] = ga[r, pl.ds(o, _LANES)]
                return c

            lax.fori_loop(0, chunk, repack, 0)
            pltpu.async_copy(
                pb, out_hbm.at[pl.ds((base + r0) * DO, chunk * DO)],
                osems[p])

        def pair(i, carry):
            half_step(i, 0)
            half_step(i, 1)
            return carry

        lax.fori_loop(0, n_pairs, pair, 0)

        for p in (0, 1):
            pltpu.make_async_copy(
                pbufs[p], out_hbm.at[pl.ds(base * DO, chunk * DO)],
                osems[p]).wait()

    return grammar_emb


def kernel(sid_idx, aid_idx, symbol_table, action_table, key_table):
    B, S = sid_idx.shape
    D = symbol_table.shape[1]
    N = B * S
    DP = (D + 7) // 8 * 8

    info = plsc.get_sparse_core_info()
    n_workers = info.num_cores * info.num_subcores
    chunk = 40

    sym_p = jnp.pad(symbol_table, ((0, 0), (0, DP - D)))
    act_p = jnp.pad(action_table, ((0, 0), (0, DP - D)))
    key_p = jnp.pad(key_table[0], (0, DP - D))

    fn = _grammar_emb_call(N, D, DP, n_workers, info.num_cores, chunk)
    out = fn(sid_idx.reshape(N), aid_idx.reshape(N), sym_p, act_p, key_p)
    return out.reshape(B, S, 3 * D)


# speedup vs baseline: 1.6154x; 1.4814x over previous
import functools

import jax
import jax.numpy as jnp
from jax import lax
from jax.experimental import pallas as pl
from jax.experimental.pallas import tpu as pltpu
from jax.experimental.pallas import tpu_sc as plsc

_L = 16


def _grammar_emb_call(B, S, D, DP, n_cores):
    FT = (3 * D + 7) // 8
    BT = B // 128
    KT0 = (2 * D) // 8
    NKT = FT - KT0
    mesh = plsc.VectorSubcoreMesh(core_axis_name="c", subcore_axis_name="s")

    @functools.partial(
        pl.kernel,
        mesh=mesh,
        out_type=jax.ShapeDtypeStruct((S * FT, BT, 1024), jnp.float32),
        scratch_types=[
            pltpu.VMEM((S, 128), jnp.int32),
            pltpu.VMEM((S, 128), jnp.int32),
            pltpu.VMEM((128, DP), jnp.float32),
            pltpu.VMEM((128, DP), jnp.float32),
            pltpu.VMEM((19, 1024), jnp.float32),
            pltpu.VMEM((19, 1024), jnp.float32),
            pltpu.VMEM((DP,), jnp.float32),
            pltpu.VMEM_SHARED((NKT, 1024), jnp.float32),
            pltpu.SemaphoreType.DMA,
            pltpu.SemaphoreType.DMA,
            pltpu.SemaphoreType.DMA,
            pltpu.SemaphoreType.DMA,
            pltpu.SemaphoreType.DMA,
        ],
        compiler_params=pltpu.CompilerParams(
            use_tc_tiling_on_sc=False, needs_layout_passes=False),
    )
    def grammar_emb(sid_hbm, aid_hbm, sym_hbm, act_hbm, key_hbm, out_hbm,
                    idx_s, idx_a, gsym, gact, slab0, slab1, kbuf, keyslab,
                    sem0, sem1, ksem, gsem0, gsem1):
        wid = lax.axis_index("s") * n_cores + lax.axis_index("c")
        sid_l = lax.axis_index("s")
        b0 = wid * 128

        pltpu.sync_copy(sid_hbm.at[:, pl.ds(b0, 128)], idx_s)
        pltpu.sync_copy(aid_hbm.at[:, pl.ds(b0, 128)], idx_a)
        pltpu.sync_copy(key_hbm, kbuf)

        lanes = lax.iota(jnp.int32, _L)
        rows = [lanes + 16 * j for j in range(8)]

        def splat_col(buf, col):
            idx = jnp.full((_L,), col, jnp.int32)
            return plsc.load_gather(buf, [idx])

        def tile_from(buf, tloc, colbase, slab):
            for fi in range(8):
                cols = jnp.full((_L,), colbase + fi, jnp.int32)
                for j in range(8):
                    v = plsc.load_gather(buf, [rows[j], cols])
                    slab[tloc, pl.ds(fi * 128 + 16 * j, _L)] = v

        half = NKT // 2

        @pl.when(sid_l == 0)
        def _():
            for h in range(2):
                def build_key(t, c, _h=h):
                    col = (_h * half + t) * 8
                    for fi in range(8):
                        v = splat_col(kbuf, col + fi)
                        for j in range(8):
                            slab0[t, pl.ds(fi * 128 + 16 * j, _L)] = v
                    return c
                lax.fori_loop(0, half, build_key, 0)
                pltpu.sync_copy(slab0, keyslab.at[pl.ds(h * half, half)])
        plsc.subcore_barrier()

        def step(s, carry):
            cp_s = pltpu.async_copy(sym_hbm.at[idx_s.at[s]], gsym, gsem0)
            cp_a = pltpu.async_copy(act_hbm.at[idx_a.at[s]], gact, gsem1)
            cp_s.wait()
            cp_a.wait()
            r0 = s * FT

            @pl.when(s >= 1)
            def _():
                pltpu.make_async_copy(
                    slab0, out_hbm.at[pl.ds(0, 19), wid, :], sem0).wait()

            def build_a(t, c):
                tile_from(gsym, t, 8 * t, slab0)
                return c
            lax.fori_loop(0, 19, build_a, 0)
            pltpu.async_copy(slab0, out_hbm.at[pl.ds(r0, 19), wid, :], sem0)

            @pl.when(s >= 1)
            def _():
                pltpu.make_async_copy(
                    slab1.at[pl.ds(0, 18)],
                    out_hbm.at[pl.ds(0, 18), wid, :], sem1).wait()

            def build_b(t, c):
                tile_from(gsym, t, 8 * (19 + t), slab1)
                return c
            lax.fori_loop(0, 18, build_b, 0)
            for fi in range(8):
                cols = jnp.full((_L,), 296 + fi if fi < 4 else fi - 4,
                                jnp.int32)
                src = gsym if fi < 4 else gact
                for j in range(8):
                    v = plsc.load_gather(src, [rows[j], cols])
                    slab1[18, pl.ds(fi * 128 + 16 * j, _L)] = v
            pltpu.async_copy(slab1, out_hbm.at[pl.ds(r0 + 19, 19), wid, :],
                             sem1)

            pltpu.make_async_copy(
                slab0, out_hbm.at[pl.ds(0, 19), wid, :], sem0).wait()

            def build_c(t, c):
                tile_from(gact, t, 8 * (38 + t) - 300, slab0)
                return c
            lax.fori_loop(0, 19, build_c, 0)
            pltpu.async_copy(slab0, out_hbm.at[pl.ds(r0 + 38, 19), wid, :],
                             sem0)

            pltpu.make_async_copy(
                slab1, out_hbm.at[pl.ds(0, 19), wid, :], sem1).wait()

            def build_d(t, c):
                tile_from(gact, t, 8 * (57 + t) - 300, slab1)
                return c
            lax.fori_loop(0, 18, build_d, 0)
            pltpu.async_copy(slab1.at[pl.ds(0, 18)],
                             out_hbm.at[pl.ds(r0 + 57, 18), wid, :], sem1)

            @pl.when(s >= 1)
            def _():
                pltpu.make_async_copy(
                    keyslab,
                    out_hbm.at[pl.ds(0, NKT), wid, :], ksem).wait()
            pltpu.async_copy(keyslab,
                             out_hbm.at[pl.ds(r0 + KT0, NKT), wid, :], ksem)
            return carry

        lax.fori_loop(0, S, step, 0)

        pltpu.make_async_copy(
            slab0, out_hbm.at[pl.ds(0, 19), wid, :], sem0).wait()
        pltpu.make_async_copy(
            slab1.at[pl.ds(0, 18)],
            out_hbm.at[pl.ds(0, 18), wid, :], sem1).wait()
        pltpu.make_async_copy(
            keyslab,
            out_hbm.at[pl.ds(0, NKT), wid, :], ksem).wait()

    return grammar_emb


def kernel(sid_idx, aid_idx, symbol_table, action_table, key_table):
    B, S = sid_idx.shape
    D = symbol_table.shape[1]
    DP = (D + 7) // 8 * 8
    FT = (3 * D + 7) // 8
    BT = B // 128

    info = plsc.get_sparse_core_info()
    sym_p = jnp.pad(symbol_table, ((0, 0), (0, DP - D)))
    act_p = jnp.pad(action_table, ((0, 0), (0, DP - D)))
    key_p = jnp.pad(key_table[0], (0, DP - D))

    fn = _grammar_emb_call(B, S, D, DP, info.num_cores)
    out = fn(sid_idx.T, aid_idx.T, sym_p, act_p, key_p)
    out = out.reshape(S, FT, BT, 8, 128)
    out = out.transpose(2, 4, 0, 1, 3).reshape(B, S, FT * 8)
    return out[:, :, :3 * D]


# speedup vs baseline: 2.5313x; 1.5670x over previous
import functools

import jax
import jax.numpy as jnp
from jax import lax
from jax.experimental import pallas as pl
from jax.experimental.pallas import tpu as pltpu
from jax.experimental.pallas import tpu_sc as plsc

_L = 16


def _grammar_emb_call(B, S, D, DP, n_cores):
    FT = (3 * D + 7) // 8
    BT = B // 128
    KT0 = (2 * D) // 8
    NKT = FT - KT0
    mesh = plsc.VectorSubcoreMesh(core_axis_name="c", subcore_axis_name="s")

    @functools.partial(
        pl.kernel,
        mesh=mesh,
        out_type=jax.ShapeDtypeStruct((S * FT, BT, 1024), jnp.float32),
        scratch_types=[
            pltpu.VMEM((S, 128), jnp.int32),
            pltpu.VMEM((S, 128), jnp.int32),
            pltpu.VMEM((128, DP), jnp.float32),
            pltpu.VMEM((128, DP), jnp.float32),
            pltpu.VMEM((18, 1024), jnp.float32),
            pltpu.VMEM((18, 1024), jnp.float32),
            pltpu.VMEM((DP,), jnp.float32),
            pltpu.VMEM_SHARED((NKT, 1024), jnp.float32),
            pltpu.SemaphoreType.DMA,
            pltpu.SemaphoreType.DMA,
            pltpu.SemaphoreType.DMA,
            pltpu.SemaphoreType.DMA,
            pltpu.SemaphoreType.DMA,
        ],
        compiler_params=pltpu.CompilerParams(
            use_tc_tiling_on_sc=False, needs_layout_passes=False),
    )
    def grammar_emb(sid_hbm, aid_hbm, sym_hbm, act_hbm, key_hbm, out_hbm,
                    idx_s, idx_a, gsym, gact, slab0, slab1, kbuf, keyslab,
                    sem0, sem1, ksem, gsem0, gsem1):
        wid = lax.axis_index("s") * n_cores + lax.axis_index("c")
        sid_l = lax.axis_index("s")
        b0 = wid * 128

        pltpu.sync_copy(sid_hbm.at[:, pl.ds(b0, 128)], idx_s)
        pltpu.sync_copy(aid_hbm.at[:, pl.ds(b0, 128)], idx_a)
        pltpu.sync_copy(key_hbm, kbuf)

        lanes = lax.iota(jnp.int32, _L)

        def rowv(j):
            return lanes + 16 * j

        def splat_col(buf, col):
            idx = jnp.full((_L,), col, jnp.int32)
            return plsc.load_gather(buf, [idx])

        def tile_from(buf, tloc, colbase, slab):
            for fi in range(8):
                cols = jnp.full((_L,), colbase + fi, jnp.int32)
                for j in range(8):
                    v = plsc.load_gather(buf, [rowv(j), cols])
                    slab[tloc, pl.ds(fi * 128 + 16 * j, _L)] = v

        def pair_from(buf, trow0, colbase, slab):
            def diag(k, carry):
                c = (lanes + k) & 15
                hi = (c >> 3) + trow0
                lo = ((c & 7) << 7) + lanes
                cols = c + colbase
                for j in range(8):
                    v = plsc.load_gather(buf, [rowv(j), cols])
                    plsc.store_scatter(slab, [hi, lo + 16 * j], v)
                return carry

            lax.fori_loop(0, 16, diag, 0)

        half = NKT // 2

        @pl.when(sid_l == 0)
        def _():
            for off, n in ((0, 18), (18, 18), (36, 2)):
                def build_key(t, c, _off=off):
                    col = (_off + t) * 8
                    for fi in range(8):
                        v = splat_col(kbuf, col + fi)
                        for j in range(8):
                            slab0[t, pl.ds(fi * 128 + 16 * j, _L)] = v
                    return c
                lax.fori_loop(0, n, build_key, 0)
                pltpu.sync_copy(slab0.at[pl.ds(0, n)],
                                keyslab.at[pl.ds(off, n)])
        plsc.subcore_barrier()

        def step(s, carry):
            cp_s = pltpu.async_copy(sym_hbm.at[idx_s.at[s]], gsym, gsem0)
            cp_a = pltpu.async_copy(act_hbm.at[idx_a.at[s]], gact, gsem1)
            cp_s.wait()
            cp_a.wait()
            r0 = s * FT

            def drain(slab, n, sem):
                pltpu.make_async_copy(
                    slab.at[pl.ds(0, n)],
                    out_hbm.at[pl.ds(0, n), wid, :], sem).wait()

            @pl.when(s >= 1)
            def _():
                drain(slab0, 3, sem0)

            def build_a(p, c):
                pair_from(gsym, 2 * p, 16 * p, slab0)
                return c
            lax.fori_loop(0, 9, build_a, 0)
            pltpu.async_copy(slab0, out_hbm.at[pl.ds(r0, 18), wid, :], sem0)

            @pl.when(s >= 1)
            def _():
                drain(slab1, 18, sem1)

            def build_b(p, c):
                pair_from(gsym, 2 * p, 144 + 16 * p, slab1)
                return c
            lax.fori_loop(0, 9, build_b, 0)
            pltpu.async_copy(slab1, out_hbm.at[pl.ds(r0 + 18, 18), wid, :],
                             sem1)

            drain(slab0, 18, sem0)
            tile_from(gsym, 0, 288, slab0)
            for fi in range(8):
                cols = jnp.full((_L,), 296 + fi if fi < 4 else fi - 4,
                                jnp.int32)
                src = gsym if fi < 4 else gact
                for j in range(8):
                    v = plsc.load_gather(src, [rowv(j), cols])
                    slab0[1, pl.ds(fi * 128 + 16 * j, _L)] = v

            def build_c(p, c):
                pair_from(gact, 2 + 2 * p, 4 + 16 * p, slab0)
                return c
            lax.fori_loop(0, 8, build_c, 0)
            pltpu.async_copy(slab0, out_hbm.at[pl.ds(r0 + 36, 18), wid, :],
                             sem0)

            drain(slab1, 18, sem1)

            def build_d(p, c):
                pair_from(gact, 2 * p, 132 + 16 * p, slab1)
                return c
            lax.fori_loop(0, 9, build_d, 0)
            pltpu.async_copy(slab1, out_hbm.at[pl.ds(r0 + 54, 18), wid, :],
                             sem1)

            drain(slab0, 18, sem0)
            pair_from(gact, 0, 276, slab0)
            tile_from(gact, 2, 292, slab0)
            pltpu.async_copy(slab0.at[pl.ds(0, 3)],
                             out_hbm.at[pl.ds(r0 + 72, 3), wid, :], sem0)

            @pl.when(s >= 1)
            def _():
                pltpu.make_async_copy(
                    keyslab,
                    out_hbm.at[pl.ds(0, NKT), wid, :], ksem).wait()
            pltpu.async_copy(keyslab,
                             out_hbm.at[pl.ds(r0 + KT0, NKT), wid, :], ksem)
            return carry

        lax.fori_loop(0, S, step, 0)

        pltpu.make_async_copy(
            slab0.at[pl.ds(0, 3)],
            out_hbm.at[pl.ds(0, 3), wid, :], sem0).wait()
        pltpu.make_async_copy(
            slab1, out_hbm.at[pl.ds(0, 18), wid, :], sem1).wait()
        pltpu.make_async_copy(
            keyslab,
            out_hbm.at[pl.ds(0, NKT), wid, :], ksem).wait()

    return grammar_emb


def kernel(sid_idx, aid_idx, symbol_table, action_table, key_table):
    B, S = sid_idx.shape
    D = symbol_table.shape[1]
    DP = (D + 7) // 8 * 8
    FT = (3 * D + 7) // 8
    BT = B // 128

    info = plsc.get_sparse_core_info()
    sym_p = jnp.pad(symbol_table, ((0, 0), (0, DP - D)))
    act_p = jnp.pad(action_table, ((0, 0), (0, DP - D)))
    key_p = jnp.pad(key_table[0], (0, DP - D))

    fn = _grammar_emb_call(B, S, D, DP, info.num_cores)
    out = fn(sid_idx.T, aid_idx.T, sym_p, act_p, key_p)
    out = out.reshape(S, FT, BT, 8, 128)
    out = out.transpose(2, 4, 0, 1, 3).reshape(B, S, FT * 8)
    return out[:, :, :3 * D]


# speedup vs baseline: 2.7132x; 1.0719x over previous
import functools

import jax
import jax.numpy as jnp
from jax import lax
from jax.experimental import pallas as pl
from jax.experimental.pallas import tpu as pltpu
from jax.experimental.pallas import tpu_sc as plsc

_L = 16


def _grammar_emb_call(B, S, D, DP, n_cores):
    FT = (3 * D + 7) // 8
    BT = B // 128
    KT0 = (2 * D) // 8
    NKT = FT - KT0
    mesh = plsc.VectorSubcoreMesh(core_axis_name="c", subcore_axis_name="s")

    @functools.partial(
        pl.kernel,
        mesh=mesh,
        out_type=jax.ShapeDtypeStruct((S * FT, BT, 1024), jnp.float32),
        scratch_types=[
            pltpu.VMEM((S, 128), jnp.int32),
            pltpu.VMEM((S, 128), jnp.int32),
            pltpu.VMEM((128, DP), jnp.float32),
            pltpu.VMEM((128, DP), jnp.float32),
            pltpu.VMEM((18, 1024), jnp.float32),
            pltpu.VMEM((18, 1024), jnp.float32),
            pltpu.VMEM((DP,), jnp.float32),
            pltpu.VMEM_SHARED((NKT, 1024), jnp.float32),
            pltpu.SemaphoreType.DMA,
            pltpu.SemaphoreType.DMA,
            pltpu.SemaphoreType.DMA,
            pltpu.SemaphoreType.DMA,
            pltpu.SemaphoreType.DMA,
        ],
        compiler_params=pltpu.CompilerParams(
            use_tc_tiling_on_sc=False, needs_layout_passes=False),
    )
    def grammar_emb(sid_hbm, aid_hbm, sym_hbm, act_hbm, key_hbm, out_hbm,
                    idx_s, idx_a, gsym, gact, slab0, slab1, kbuf, keyslab,
                    sem0, sem1, ksem, gsem0, gsem1):
        wid = lax.axis_index("s") * n_cores + lax.axis_index("c")
        sid_l = lax.axis_index("s")
        b0 = wid * 128

        pltpu.sync_copy(sid_hbm.at[:, pl.ds(b0, 128)], idx_s)
        pltpu.sync_copy(aid_hbm.at[:, pl.ds(b0, 128)], idx_a)
        pltpu.sync_copy(key_hbm, kbuf)

        lanes = lax.iota(jnp.int32, _L)

        def rowv(j):
            return lanes + 16 * j

        def splat_col(buf, col):
            idx = jnp.full((_L,), col, jnp.int32)
            return plsc.load_gather(buf, [idx])

        def tile_from(buf, tloc, colbase, slab):
            for fi in range(8):
                cols = jnp.full((_L,), colbase + fi, jnp.int32)
                for j in range(8):
                    v = plsc.load_gather(buf, [rowv(j), cols])
                    slab[tloc, pl.ds(fi * 128 + 16 * j, _L)] = v

        def pair_from(buf, trow0, colbase, slab):
            def diag(i, carry):
                for k in (2 * i, 2 * i + 1):
                    c = (lanes + k) & 15
                    hi = (c >> 3) + trow0
                    lo = ((c & 7) << 7) + lanes
                    cols = c + colbase
                    for j in range(8):
                        v = plsc.load_gather(buf, [rowv(j), cols])
                        plsc.store_scatter(slab, [hi, lo + 16 * j], v)
                return carry

            lax.fori_loop(0, 8, diag, 0)

        half = NKT // 2

        @pl.when(sid_l == 0)
        def _():
            for off, n in ((0, 18), (18, 18), (36, 2)):
                def build_key(t, c, _off=off):
                    col = (_off + t) * 8
                    for fi in range(8):
                        v = splat_col(kbuf, col + fi)
                        for j in range(8):
                            slab0[t, pl.ds(fi * 128 + 16 * j, _L)] = v
                    return c
                lax.fori_loop(0, n, build_key, 0)
                pltpu.sync_copy(slab0.at[pl.ds(0, n)],
                                keyslab.at[pl.ds(off, n)])
        plsc.subcore_barrier()

        pltpu.async_copy(sym_hbm.at[idx_s.at[0]], gsym, gsem0)
        pltpu.async_copy(act_hbm.at[idx_a.at[0]], gact, gsem1)

        def step(s, carry):
            pltpu.make_async_copy(
                sym_hbm.at[idx_s.at[s]], gsym, gsem0).wait()
            pltpu.make_async_copy(
                act_hbm.at[idx_a.at[s]], gact, gsem1).wait()
            r0 = s * FT

            def drain(slab, n, sem):
                pltpu.make_async_copy(
                    slab.at[pl.ds(0, n)],
                    out_hbm.at[pl.ds(0, n), wid, :], sem).wait()

            @pl.when(s >= 1)
            def _():
                drain(slab0, 3, sem0)

            def build_a(p, c):
                pair_from(gsym, 2 * p, 16 * p, slab0)
                return c
            lax.fori_loop(0, 9, build_a, 0)
            pltpu.async_copy(slab0, out_hbm.at[pl.ds(r0, 18), wid, :], sem0)

            @pl.when(s >= 1)
            def _():
                drain(slab1, 18, sem1)

            def build_b(p, c):
                pair_from(gsym, 2 * p, 144 + 16 * p, slab1)
                return c
            lax.fori_loop(0, 9, build_b, 0)
            pltpu.async_copy(slab1, out_hbm.at[pl.ds(r0 + 18, 18), wid, :],
                             sem1)

            drain(slab0, 18, sem0)
            tile_from(gsym, 0, 288, slab0)
            for fi in range(8):
                cols = jnp.full((_L,), 296 + fi if fi < 4 else fi - 4,
                                jnp.int32)
                src = gsym if fi < 4 else gact
                for j in range(8):
                    v = plsc.load_gather(src, [rowv(j), cols])
                    slab0[1, pl.ds(fi * 128 + 16 * j, _L)] = v

            def build_c(p, c):
                pair_from(gact, 2 + 2 * p, 4 + 16 * p, slab0)
                return c
            lax.fori_loop(0, 8, build_c, 0)
            pltpu.async_copy(slab0, out_hbm.at[pl.ds(r0 + 36, 18), wid, :],
                             sem0)

            @pl.when(s + 1 < S)
            def _():
                pltpu.async_copy(sym_hbm.at[idx_s.at[s + 1]], gsym, gsem0)

            drain(slab1, 18, sem1)

            def build_d(p, c):
                pair_from(gact, 2 * p, 132 + 16 * p, slab1)
                return c
            lax.fori_loop(0, 9, build_d, 0)
            pltpu.async_copy(slab1, out_hbm.at[pl.ds(r0 + 54, 18), wid, :],
                             sem1)

            drain(slab0, 18, sem0)
            pair_from(gact, 0, 276, slab0)
            tile_from(gact, 2, 292, slab0)
            pltpu.async_copy(slab0.at[pl.ds(0, 3)],
                             out_hbm.at[pl.ds(r0 + 72, 3), wid, :], sem0)

            @pl.when(s + 1 < S)
            def _():
                pltpu.async_copy(act_hbm.at[idx_a.at[s + 1]], gact, gsem1)

            @pl.when(s >= 1)
            def _():
                pltpu.make_async_copy(
                    keyslab,
                    out_hbm.at[pl.ds(0, NKT), wid, :], ksem).wait()
            pltpu.async_copy(keyslab,
                             out_hbm.at[pl.ds(r0 + KT0, NKT), wid, :], ksem)
            return carry

        lax.fori_loop(0, S, step, 0)

        pltpu.make_async_copy(
            slab0.at[pl.ds(0, 3)],
            out_hbm.at[pl.ds(0, 3), wid, :], sem0).wait()
        pltpu.make_async_copy(
            slab1, out_hbm.at[pl.ds(0, 18), wid, :], sem1).wait()
        pltpu.make_async_copy(
            keyslab,
            out_hbm.at[pl.ds(0, NKT), wid, :], ksem).wait()

    return grammar_emb


def kernel(sid_idx, aid_idx, symbol_table, action_table, key_table):
    B, S = sid_idx.shape
    D = symbol_table.shape[1]
    DP = (D + 7) // 8 * 8
    FT = (3 * D + 7) // 8
    BT = B // 128

    info = plsc.get_sparse_core_info()
    sym_p = jnp.pad(symbol_table, ((0, 0), (0, DP - D)))
    act_p = jnp.pad(action_table, ((0, 0), (0, DP - D)))
    key_p = jnp.pad(key_table[0], (0, DP - D))

    fn = _grammar_emb_call(B, S, D, DP, info.num_cores)
    out = fn(sid_idx.T, aid_idx.T, sym_p, act_p, key_p)
    out = out.reshape(S, FT, BT, 8, 128)
    out = out.transpose(2, 4, 0, 1, 3).reshape(B, S, FT * 8)
    return out[:, :, :3 * D]


# speedup vs baseline: 2.7763x; 1.0233x over previous
import functools

import jax
import jax.numpy as jnp
from jax import lax
from jax.experimental import pallas as pl
from jax.experimental.pallas import tpu as pltpu
from jax.experimental.pallas import tpu_sc as plsc

_L = 16


def _grammar_emb_call(B, S, D, DP, n_cores):
    FT = (3 * D + 7) // 8
    BT = B // 128
    KT0 = (2 * D) // 8
    NKT = FT - KT0
    mesh = plsc.VectorSubcoreMesh(core_axis_name="c", subcore_axis_name="s")

    @functools.partial(
        pl.kernel,
        mesh=mesh,
        out_type=jax.ShapeDtypeStruct((S * FT, BT, 1024), jnp.float32),
        scratch_types=[
            pltpu.VMEM((S, 128), jnp.int32),
            pltpu.VMEM((S, 128), jnp.int32),
            pltpu.VMEM((128, DP), jnp.float32),
            pltpu.VMEM((128, DP), jnp.float32),
            pltpu.VMEM((18, 1024), jnp.float32),
            pltpu.VMEM((18, 1024), jnp.float32),
            pltpu.VMEM((DP,), jnp.float32),
            pltpu.VMEM_SHARED((NKT, 1024), jnp.float32),
            pltpu.SemaphoreType.DMA,
            pltpu.SemaphoreType.DMA,
            pltpu.SemaphoreType.DMA,
            pltpu.SemaphoreType.DMA,
            pltpu.SemaphoreType.DMA,
        ],
        compiler_params=pltpu.CompilerParams(
            use_tc_tiling_on_sc=False, needs_layout_passes=False),
    )
    def grammar_emb(sid_hbm, aid_hbm, sym_hbm, act_hbm, key_hbm, out_hbm,
                    idx_s, idx_a, gsym, gact, slab0, slab1, kbuf, keyslab,
                    sem0, sem1, ksem, gsem0, gsem1):
        wid = lax.axis_index("s") * n_cores + lax.axis_index("c")
        sid_l = lax.axis_index("s")
        b0 = wid * 128

        pltpu.sync_copy(sid_hbm.at[:, pl.ds(b0, 128)], idx_s)
        pltpu.sync_copy(aid_hbm.at[:, pl.ds(b0, 128)], idx_a)
        pltpu.sync_copy(key_hbm, kbuf)

        lanes = lax.iota(jnp.int32, _L)

        def rowv(j):
            return lanes + 16 * j

        def splat_col(buf, col):
            idx = jnp.full((_L,), col, jnp.int32)
            return plsc.load_gather(buf, [idx])

        def tile_from(buf, tloc, colbase, slab):
            for fi in range(8):
                cols = jnp.full((_L,), colbase + fi, jnp.int32)
                for j in range(8):
                    v = plsc.load_gather(buf, [rowv(j), cols])
                    slab[tloc, pl.ds(fi * 128 + 16 * j, _L)] = v

        def pair_from(buf, trow0, colbase, slab):
            def diag(i, carry):
                for k in (4 * i, 4 * i + 1, 4 * i + 2, 4 * i + 3):
                    c = (lanes + k) & 15
                    hi = (c >> 3) + trow0
                    lo = ((c & 7) << 7) + lanes
                    cols = c + colbase
                    for j in range(8):
                        v = plsc.load_gather(buf, [rowv(j), cols])
                        plsc.store_scatter(slab, [hi, lo + 16 * j], v)
                return carry

            lax.fori_loop(0, 4, diag, 0)

        half = NKT // 2

        @pl.when(sid_l == 0)
        def _():
            for off, n in ((0, 18), (18, 18), (36, 2)):
                def build_key(t, c, _off=off):
                    col = (_off + t) * 8
                    for fi in range(8):
                        v = splat_col(kbuf, col + fi)
                        for j in range(8):
                            slab0[t, pl.ds(fi * 128 + 16 * j, _L)] = v
                    return c
                lax.fori_loop(0, n, build_key, 0)
                pltpu.sync_copy(slab0.at[pl.ds(0, n)],
                                keyslab.at[pl.ds(off, n)])
        plsc.subcore_barrier()

        pltpu.async_copy(sym_hbm.at[idx_s.at[0]], gsym, gsem0)
        pltpu.async_copy(act_hbm.at[idx_a.at[0]], gact, gsem1)

        def step(s, carry):
            pltpu.make_async_copy(
                sym_hbm.at[idx_s.at[s]], gsym, gsem0).wait()
            pltpu.make_async_copy(
                act_hbm.at[idx_a.at[s]], gact, gsem1).wait()
            r0 = s * FT

            def drain(slab, n, sem):
                pltpu.make_async_copy(
                    slab.at[pl.ds(0, n)],
                    out_hbm.at[pl.ds(0, n), wid, :], sem).wait()

            @pl.when(s >= 1)
            def _():
                drain(slab0, 3, sem0)

            def build_a(p, c):
                pair_from(gsym, 2 * p, 16 * p, slab0)
                return c
            lax.fori_loop(0, 9, build_a, 0)
            pltpu.async_copy(slab0, out_hbm.at[pl.ds(r0, 18), wid, :], sem0)

            @pl.when(s >= 1)
            def _():
                drain(slab1, 18, sem1)

            def build_b(p, c):
                pair_from(gsym, 2 * p, 144 + 16 * p, slab1)
                return c
            lax.fori_loop(0, 9, build_b, 0)
            pltpu.async_copy(slab1, out_hbm.at[pl.ds(r0 + 18, 18), wid, :],
                             sem1)

            drain(slab0, 18, sem0)
            tile_from(gsym, 0, 288, slab0)
            for fi in range(8):
                cols = jnp.full((_L,), 296 + fi if fi < 4 else fi - 4,
                                jnp.int32)
                src = gsym if fi < 4 else gact
                for j in range(8):
                    v = plsc.load_gather(src, [rowv(j), cols])
                    slab0[1, pl.ds(fi * 128 + 16 * j, _L)] = v

            def build_c(p, c):
                pair_from(gact, 2 + 2 * p, 4 + 16 * p, slab0)
                return c
            lax.fori_loop(0, 8, build_c, 0)
            pltpu.async_copy(slab0, out_hbm.at[pl.ds(r0 + 36, 18), wid, :],
                             sem0)

            @pl.when(s + 1 < S)
            def _():
                pltpu.async_copy(sym_hbm.at[idx_s.at[s + 1]], gsym, gsem0)

            drain(slab1, 18, sem1)

            def build_d(p, c):
                pair_from(gact, 2 * p, 132 + 16 * p, slab1)
                return c
            lax.fori_loop(0, 9, build_d, 0)
            pltpu.async_copy(slab1, out_hbm.at[pl.ds(r0 + 54, 18), wid, :],
                             sem1)

            drain(slab0, 18, sem0)
            pair_from(gact, 0, 276, slab0)
            tile_from(gact, 2, 292, slab0)
            pltpu.async_copy(slab0.at[pl.ds(0, 3)],
                             out_hbm.at[pl.ds(r0 + 72, 3), wid, :], sem0)

            @pl.when(s + 1 < S)
            def _():
                pltpu.async_copy(act_hbm.at[idx_a.at[s + 1]], gact, gsem1)

            @pl.when(s >= 1)
            def _():
                pltpu.make_async_copy(
                    keyslab,
                    out_hbm.at[pl.ds(0, NKT), wid, :], ksem).wait()
            pltpu.async_copy(keyslab,
                             out_hbm.at[pl.ds(r0 + KT0, NKT), wid, :], ksem)
            return carry

        lax.fori_loop(0, S, step, 0)

        pltpu.make_async_copy(
            slab0.at[pl.ds(0, 3)],
            out_hbm.at[pl.ds(0, 3), wid, :], sem0).wait()
        pltpu.make_async_copy(
            slab1, out_hbm.at[pl.ds(0, 18), wid, :], sem1).wait()
        pltpu.make_async_copy(
            keyslab,
            out_hbm.at[pl.ds(0, NKT), wid, :], ksem).wait()

    return grammar_emb


def kernel(sid_idx, aid_idx, symbol_table, action_table, key_table):
    B, S = sid_idx.shape
    D = symbol_table.shape[1]
    DP = (D + 7) // 8 * 8
    FT = (3 * D + 7) // 8
    BT = B // 128

    info = plsc.get_sparse_core_info()
    sym_p = jnp.pad(symbol_table, ((0, 0), (0, DP - D)))
    act_p = jnp.pad(action_table, ((0, 0), (0, DP - D)))
    key_p = jnp.pad(key_table[0], (0, DP - D))

    fn = _grammar_emb_call(B, S, D, DP, info.num_cores)
    out = fn(sid_idx.T, aid_idx.T, sym_p, act_p, key_p)
    out = out.reshape(S, FT, BT, 8, 128)
    out = out.transpose(2, 4, 0, 1, 3).reshape(B, S, FT * 8)
    return out[:, :, :3 * D]
